# Initial kernel scaffold; baseline (speedup 1.0000x reference)
import jax
import jax.numpy as jnp
from jax.experimental import pallas as pl


def kernel(feat, cluster0, cluster1, cluster2, neighbor_idx, W_proj0, g_proj0, b_proj0, W_proj1, g_proj1, b_proj1, W_proj2, g_proj2, b_proj2, W_proj3, g_proj3, b_proj3, W_lw0, g_lw0, b_lw0, W_wt0, W_lw1, g_lw1, b_lw1, W_wt1, W_lw2, g_lw2, b_lw2, W_wt2, W_adp, W_fuse, g_fuse, b_fuse, W_conv1, W_conv2, g_n1, b_n1, g_n2, b_n2):
    raise NotImplementedError("write your pallas kernel here")


# speedup vs baseline: 1.8243x; 1.8243x over previous
import functools

import jax
import jax.numpy as jnp
import numpy as np
from jax import lax
from jax.experimental import pallas as pl
from jax.experimental.pallas import tpu as pltpu
from jax.experimental.pallas import tpu_sc as plsc

N = 20000
C = 128
K = 27
NSEG = (2048, 512, 128)
B = 800
NB = N // B
FN = float(N)
EPS = 1e-5

NW = 32
PW = 632
N_PAD = NW * PW
CH = 8
NCH = PW // CH


def _bn_affine(S, mu_sum, W, g, b):
    mu = mu_sum / FN
    m = jnp.dot(mu, W, preferred_element_type=jnp.float32)
    SW = jnp.dot(S, W, preferred_element_type=jnp.float32)
    ey2 = jnp.sum(W * SW, axis=0, keepdims=True) / FN
    var = ey2 - m * m
    alpha = g * lax.rsqrt(var + EPS)
    return W * alpha, b - m * alpha


def _oh_t(cl_row, nseg):
    iota = lax.broadcasted_iota(jnp.int32, (nseg, B), 0)
    return (iota == cl_row).astype(jnp.float32)


def _oh(cl_col, nseg):
    iota = lax.broadcasted_iota(jnp.int32, (B, nseg), 1)
    return (iota == cl_col).astype(jnp.float32)


def _gram_body(feat_ref, s_ref, mu_ref):
    @pl.when(pl.program_id(0) == 0)
    def _():
        s_ref[...] = jnp.zeros_like(s_ref)
        mu_ref[...] = jnp.zeros_like(mu_ref)

    x = feat_ref[...]
    s_ref[...] += lax.dot_general(x, x, (((0,), (0,)), ((), ())),
                                  preferred_element_type=jnp.float32)
    mu_ref[...] += jnp.sum(x, axis=0, keepdims=True)


def _gram(feat):
    return pl.pallas_call(
        _gram_body,
        grid=(NB,),
        in_specs=[pl.BlockSpec((B, C), lambda i: (i, 0))],
        out_specs=[pl.BlockSpec((C, C), lambda i: (0, 0)),
                   pl.BlockSpec((1, C), lambda i: (0, 0))],
        out_shape=[jax.ShapeDtypeStruct((C, C), jnp.float32),
                   jax.ShapeDtypeStruct((1, C), jnp.float32)],
    )(feat)


def _lw_body(nseg, feat_ref, clt_ref, s_ref, mu_ref, w_ref, g_ref, b_ref,
             pw_ref, ss_ref, cnt_ref):
    @pl.when(pl.program_id(0) == 0)
    def _():
        ss_ref[...] = jnp.zeros_like(ss_ref)
        cnt_ref[...] = jnp.zeros_like(cnt_ref)

    Wa, beta = _bn_affine(s_ref[...], mu_ref[...], w_ref[...],
                          g_ref[...], b_ref[...])
    pw = jnp.maximum(
        jnp.dot(feat_ref[...], Wa, preferred_element_type=jnp.float32) + beta,
        0.0)
    pw_ref[...] = pw
    oh_t = _oh_t(clt_ref[0], nseg)
    ss_ref[...] += jnp.dot(oh_t, pw, preferred_element_type=jnp.float32)
    cnt_ref[...] += jnp.sum(oh_t, axis=1, keepdims=True)


def _lw(feat, clt, S, mu, W, g, b, nseg):
    return pl.pallas_call(
        functools.partial(_lw_body, nseg),
        grid=(NB,),
        in_specs=[pl.BlockSpec((B, C), lambda i: (i, 0)),
                  pl.BlockSpec((1, 1, B), lambda i: (i, 0, 0)),
                  pl.BlockSpec((C, C), lambda i: (0, 0)),
                  pl.BlockSpec((1, C), lambda i: (0, 0)),
                  pl.BlockSpec((C, C), lambda i: (0, 0)),
                  pl.BlockSpec((1, C), lambda i: (0, 0)),
                  pl.BlockSpec((1, C), lambda i: (0, 0))],
        out_specs=[pl.BlockSpec((B, C), lambda i: (i, 0)),
                   pl.BlockSpec((nseg, C), lambda i: (0, 0)),
                   pl.BlockSpec((nseg, 1), lambda i: (0, 0))],
        out_shape=[jax.ShapeDtypeStruct((N, C), jnp.float32),
                   jax.ShapeDtypeStruct((nseg, C), jnp.float32),
                   jax.ShapeDtypeStruct((nseg, 1), jnp.float32)],
    )(feat, clt, S, mu, W, g, b)


def _subwt_body(nseg, pw_ref, clc_ref, ss_ref, cnt_ref, wt_ref, y_ref, mx_ref):
    @pl.when(pl.program_id(0) == 0)
    def _():
        mx_ref[...] = jnp.full_like(mx_ref, -jnp.inf)

    mean_tab = ss_ref[...] / jnp.maximum(cnt_ref[...], 1.0)
    oh = _oh(clc_ref[...], nseg)
    pwc = pw_ref[...] - jnp.dot(oh, mean_tab,
                                preferred_element_type=jnp.float32)
    y = jnp.dot(pwc, wt_ref[...], preferred_element_type=jnp.float32)
    y_ref[...] = y
    mx_ref[...] = jnp.maximum(mx_ref[...], jnp.max(y, axis=0, keepdims=True))


def _subwt(pw, clc, ss, cnt, wt, nseg):
    return pl.pallas_call(
        functools.partial(_subwt_body, nseg),
        grid=(NB,),
        in_specs=[pl.BlockSpec((B, C), lambda i: (i, 0)),
                  pl.BlockSpec((B, 1), lambda i: (i, 0)),
                  pl.BlockSpec((nseg, C), lambda i: (0, 0)),
                  pl.BlockSpec((nseg, 1), lambda i: (0, 0)),
                  pl.BlockSpec((C, C), lambda i: (0, 0))],
        out_specs=[pl.BlockSpec((B, C), lambda i: (i, 0)),
                   pl.BlockSpec((1, C), lambda i: (0, 0))],
        out_shape=[jax.ShapeDtypeStruct((N, C), jnp.float32),
                   jax.ShapeDtypeStruct((1, C), jnp.float32)],
    )(pw, clc, ss, cnt, wt)


def _exp_body(nseg, y_ref, mx_ref, clt_ref, e_ref, es_ref):
    @pl.when(pl.program_id(0) == 0)
    def _():
        es_ref[...] = jnp.zeros_like(es_ref)

    m = jnp.max(mx_ref[...])
    e = jnp.exp(y_ref[...] - m)
    e_ref[...] = e
    oh_t = _oh_t(clt_ref[0], nseg)
    es_ref[...] += jnp.dot(oh_t, e, preferred_element_type=jnp.float32)


def _expseg(y, mx, clt, nseg):
    return pl.pallas_call(
        functools.partial(_exp_body, nseg),
        grid=(NB,),
        in_specs=[pl.BlockSpec((B, C), lambda i: (i, 0)),
                  pl.BlockSpec((1, C), lambda i: (0, 0)),
                  pl.BlockSpec((1, 1, B), lambda i: (i, 0, 0))],
        out_specs=[pl.BlockSpec((B, C), lambda i: (i, 0)),
                   pl.BlockSpec((nseg, C), lambda i: (0, 0))],
        out_shape=[jax.ShapeDtypeStruct((N, C), jnp.float32),
                   jax.ShapeDtypeStruct((nseg, C), jnp.float32)],
    )(y, mx, clt)


def _pf_body(nseg, e_ref, es_ref, clt_ref, clc_ref, feat_ref, s_ref, mu_ref,
             wp_ref, g_ref, b_ref, ps_ref):
    @pl.when(pl.program_id(0) == 0)
    def _():
        ps_ref[...] = jnp.zeros_like(ps_ref)

    Wa, beta = _bn_affine(s_ref[...], mu_ref[...], wp_ref[...],
                          g_ref[...], b_ref[...])
    proj = jnp.maximum(
        jnp.dot(feat_ref[...], Wa, preferred_element_type=jnp.float32) + beta,
        0.0)
    oh = _oh(clc_ref[...], nseg)
    denom = jnp.dot(oh, es_ref[...], preferred_element_type=jnp.float32) + 1e-6
    p = proj * (e_ref[...] / denom)
    oh_t = _oh_t(clt_ref[0], nseg)
    ps_ref[...] += jnp.dot(oh_t, p, preferred_element_type=jnp.float32)


def _pf(e, es, clt, clc, feat, S, mu, Wp, g, b, nseg):
    return pl.pallas_call(
        functools.partial(_pf_body, nseg),
        grid=(NB,),
        in_specs=[pl.BlockSpec((B, C), lambda i: (i, 0)),
                  pl.BlockSpec((nseg, C), lambda i: (0, 0)),
                  pl.BlockSpec((1, 1, B), lambda i: (i, 0, 0)),
                  pl.BlockSpec((B, 1), lambda i: (i, 0)),
                  pl.BlockSpec((B, C), lambda i: (i, 0)),
                  pl.BlockSpec((C, C), lambda i: (0, 0)),
                  pl.BlockSpec((1, C), lambda i: (0, 0)),
                  pl.BlockSpec((C, C), lambda i: (0, 0)),
                  pl.BlockSpec((1, C), lambda i: (0, 0)),
                  pl.BlockSpec((1, C), lambda i: (0, 0))],
        out_specs=[pl.BlockSpec((nseg, C), lambda i: (0, 0))],
        out_shape=[jax.ShapeDtypeStruct((nseg, C), jnp.float32)],
    )(e, es, clt, clc, feat, S, mu, Wp, g, b)[0]


def _fuse1_body(feat_ref, cl0_ref, cl1_ref, cl2_ref, ps0_ref, ps1_ref,
                ps2_ref, s_ref, mu_ref, wp3_ref, g3_ref, b3_ref, wadp_ref,
                wft_ref, wfb_ref, y_ref, st_ref):
    @pl.when(pl.program_id(0) == 0)
    def _():
        st_ref[...] = jnp.zeros_like(st_ref)

    Wa, beta = _bn_affine(s_ref[...], mu_ref[...], wp3_ref[...],
                          g3_ref[...], b3_ref[...])
    feat = feat_ref[...]
    featp = jnp.maximum(
        jnp.dot(feat, Wa, preferred_element_type=jnp.float32) + beta, 0.0)

    ya = jnp.dot(feat, wadp_ref[...], preferred_element_type=jnp.float32)
    lane = lax.broadcasted_iota(jnp.int32, (B, C), 1)
    ym = jnp.where(lane < 3, ya, -jnp.inf)
    mxr = jnp.max(ym, axis=1, keepdims=True)
    ee = jnp.exp(ym - mxr)
    adp = ee / jnp.sum(ee, axis=1, keepdims=True)

    acc = jnp.zeros((B, C), jnp.float32)
    for i, (cl_ref, ps_ref, nseg) in enumerate(
            ((cl0_ref, ps0_ref, NSEG[0]),
             (cl1_ref, ps1_ref, NSEG[1]),
             (cl2_ref, ps2_ref, NSEG[2]))):
        oh = _oh(cl_ref[...], nseg)
        pf = jnp.dot(oh, ps_ref[...], preferred_element_type=jnp.float32)
        acc = acc + adp[:, i:i + 1] * pf

    y = (jnp.dot(featp, wft_ref[...], preferred_element_type=jnp.float32)
         + jnp.dot(acc, wfb_ref[...], preferred_element_type=jnp.float32))
    y_ref[...] = y
    st_ref[...] += jnp.concatenate(
        [jnp.sum(y, axis=0, keepdims=True),
         jnp.sum(y * y, axis=0, keepdims=True)], axis=0)


def _fuse1(feat, cl0c, cl1c, cl2c, ps0, ps1, ps2, S, mu, Wp3, g3, b3,
           WadpP, Wft, Wfb):
    return pl.pallas_call(
        _fuse1_body,
        grid=(NB,),
        in_specs=[pl.BlockSpec((B, C), lambda i: (i, 0)),
                  pl.BlockSpec((B, 1), lambda i: (i, 0)),
                  pl.BlockSpec((B, 1), lambda i: (i, 0)),
                  pl.BlockSpec((B, 1), lambda i: (i, 0)),
                  pl.BlockSpec((NSEG[0], C), lambda i: (0, 0)),
                  pl.BlockSpec((NSEG[1], C), lambda i: (0, 0)),
                  pl.BlockSpec((NSEG[2], C), lambda i: (0, 0)),
                  pl.BlockSpec((C, C), lambda i: (0, 0)),
                  pl.BlockSpec((1, C), lambda i: (0, 0)),
                  pl.BlockSpec((C, C), lambda i: (0, 0)),
                  pl.BlockSpec((1, C), lambda i: (0, 0)),
                  pl.BlockSpec((1, C), lambda i: (0, 0)),
                  pl.BlockSpec((C, C), lambda i: (0, 0)),
                  pl.BlockSpec((C, C), lambda i: (0, 0)),
                  pl.BlockSpec((C, C), lambda i: (0, 0))],
        out_specs=[pl.BlockSpec((B, C), lambda i: (i, 0)),
                   pl.BlockSpec((2, C), lambda i: (0, 0))],
        out_shape=[jax.ShapeDtypeStruct((N, C), jnp.float32),
                   jax.ShapeDtypeStruct((2, C), jnp.float32)],
    )(feat, cl0c, cl1c, cl2c, ps0, ps1, ps2, S, mu, Wp3, g3, b3,
      WadpP, Wft, Wfb)


def _fuse2_body(y_ref, st_ref, g_ref, b_ref, feat_ref, h_ref):
    st = st_ref[...]
    m = st[0:1] / FN
    var = st[1:2] / FN - m * m
    h = jnp.maximum(
        (y_ref[...] - m) * lax.rsqrt(var + EPS) * g_ref[...] + b_ref[...],
        0.0) + feat_ref[...]
    h_ref[...] = h


def _fuse2(y, st, g, b, feat):
    return pl.pallas_call(
        _fuse2_body,
        grid=(NB,),
        in_specs=[pl.BlockSpec((B, C), lambda i: (i, 0)),
                  pl.BlockSpec((2, C), lambda i: (0, 0)),
                  pl.BlockSpec((1, C), lambda i: (0, 0)),
                  pl.BlockSpec((1, C), lambda i: (0, 0)),
                  pl.BlockSpec((B, C), lambda i: (i, 0))],
        out_specs=[pl.BlockSpec((B, C), lambda i: (i, 0))],
        out_shape=[jax.ShapeDtypeStruct((N, C), jnp.float32)],
    )(y, st, g, b, feat)[0]


def _zmm_plain_body(x_ref, wd_ref, z_ref):
    z_ref[...] = jnp.dot(x_ref[...], wd_ref[0],
                         preferred_element_type=jnp.float32)


def _zmm_plain(x, Wd):
    return pl.pallas_call(
        _zmm_plain_body,
        grid=(NB, K),
        in_specs=[pl.BlockSpec((B, C), lambda b, k: (b, 0)),
                  pl.BlockSpec((1, C, C), lambda b, k: (k, 0, 0))],
        out_specs=[pl.BlockSpec((B, C), lambda b, k: (k * NB + b, 0))],
        out_shape=[jax.ShapeDtypeStruct((K * N, C), jnp.float32)],
    )(x, Wd)[0]


def _zmm_bn_body(x_ref, wd_ref, st_ref, g_ref, b_ref, z_ref):
    st = st_ref[...]
    m = st[0:1] / FN
    var = st[1:2] / FN - m * m
    alpha = g_ref[...] * lax.rsqrt(var + EPS)
    beta = b_ref[...] - m * alpha
    bnx = x_ref[...] * alpha + beta
    z_ref[...] = jnp.dot(bnx, wd_ref[0], preferred_element_type=jnp.float32)


def _zmm_bn(x, Wd, st, g, b):
    return pl.pallas_call(
        _zmm_bn_body,
        grid=(NB, K),
        in_specs=[pl.BlockSpec((B, C), lambda b, k: (b, 0)),
                  pl.BlockSpec((1, C, C), lambda b, k: (k, 0, 0)),
                  pl.BlockSpec((2, C), lambda b, k: (0, 0)),
                  pl.BlockSpec((1, C), lambda b, k: (0, 0)),
                  pl.BlockSpec((1, C), lambda b, k: (0, 0))],
        out_specs=[pl.BlockSpec((B, C), lambda b, k: (k * NB + b, 0))],
        out_shape=[jax.ShapeDtypeStruct((K * N, C), jnp.float32)],
    )(x, Wd, st, g, b)[0]


def _stats_body(x_ref, st_ref):
    @pl.when(pl.program_id(0) == 0)
    def _():
        st_ref[...] = jnp.zeros_like(st_ref)

    x = x_ref[...]
    st_ref[...] += jnp.concatenate(
        [jnp.sum(x, axis=0, keepdims=True),
         jnp.sum(x * x, axis=0, keepdims=True)], axis=0)


def _stats(x):
    return pl.pallas_call(
        _stats_body,
        grid=(NB,),
        in_specs=[pl.BlockSpec((B, C), lambda i: (i, 0))],
        out_specs=[pl.BlockSpec((2, C), lambda i: (0, 0))],
        out_shape=[jax.ShapeDtypeStruct((2, C), jnp.float32)],
    )(x)[0]


def _final_body(x_ref, st_ref, g_ref, b_ref, h_ref, o_ref):
    st = st_ref[...]
    m = st[0:1] / FN
    var = st[1:2] / FN - m * m
    o_ref[...] = jnp.maximum(
        (x_ref[...] - m) * lax.rsqrt(var + EPS) * g_ref[...] + b_ref[...]
        + h_ref[...], 0.0)


def _final(x, st, g, b, h):
    return pl.pallas_call(
        _final_body,
        grid=(NB,),
        in_specs=[pl.BlockSpec((B, C), lambda i: (i, 0)),
                  pl.BlockSpec((2, C), lambda i: (0, 0)),
                  pl.BlockSpec((1, C), lambda i: (0, 0)),
                  pl.BlockSpec((1, C), lambda i: (0, 0)),
                  pl.BlockSpec((B, C), lambda i: (i, 0))],
        out_specs=[pl.BlockSpec((B, C), lambda i: (i, 0))],
        out_shape=[jax.ShapeDtypeStruct((N, C), jnp.float32)],
    )(x, st, g, b, h)[0]


def _sc_bagsum_body(z_hbm, idx_hbm, out_hbm, idx_v, rows_v, acc_v, sem):
    cid = lax.axis_index("c")
    sid = lax.axis_index("s")
    wid = sid * 2 + cid
    base = wid * PW

    def chunk(c, carry):
        n0 = base + c * CH
        pltpu.sync_copy(idx_hbm.at[pl.ds(n0 * K, CH * K)], idx_v)
        pltpu.async_copy(z_hbm.at[idx_v], rows_v, sem).wait()
        for p in range(CH):
            accs = tuple(rows_v[p * K, pl.ds(16 * j, 16)] for j in range(8))

            def kstep(kk, a):
                return tuple(a[j] + rows_v[p * K + kk, pl.ds(16 * j, 16)]
                             for j in range(8))

            accs = lax.fori_loop(1, K, kstep, accs)
            for j in range(8):
                acc_v[p, pl.ds(16 * j, 16)] = accs[j]
        pltpu.sync_copy(acc_v, out_hbm.at[pl.ds(n0, CH)])
        return carry

    lax.fori_loop(0, NCH, chunk, 0)


def _sc_bagsum(z_flat, idx_flat):
    mesh = plsc.VectorSubcoreMesh(core_axis_name="c", subcore_axis_name="s",
                                  num_cores=2, num_subcores=16)
    f = pl.kernel(
        _sc_bagsum_body,
        out_type=jax.ShapeDtypeStruct((N_PAD, C), jnp.float32),
        mesh=mesh,
        scratch_types=[pltpu.VMEM((CH * K,), jnp.int32),
                       pltpu.VMEM((CH * K, C), jnp.float32),
                       pltpu.VMEM((CH, C), jnp.float32),
                       pltpu.SemaphoreType.DMA],
    )
    return f(z_flat, idx_flat)


def kernel(feat, cluster0, cluster1, cluster2, neighbor_idx,
           W_proj0, g_proj0, b_proj0, W_proj1, g_proj1, b_proj1,
           W_proj2, g_proj2, b_proj2, W_proj3, g_proj3, b_proj3,
           W_lw0, g_lw0, b_lw0, W_wt0, W_lw1, g_lw1, b_lw1, W_wt1,
           W_lw2, g_lw2, b_lw2, W_wt2, W_adp, W_fuse, g_fuse, b_fuse,
           W_conv1, W_conv2, g_n1, b_n1, g_n2, b_n2):
    r1 = lambda v: v.reshape(1, C)
    clusters = (cluster0.astype(jnp.int32), cluster1.astype(jnp.int32),
                cluster2.astype(jnp.int32))
    clt = [cl.reshape(NB, 1, B) for cl in clusters]
    clc = [cl.reshape(N, 1) for cl in clusters]

    W_lws = (W_lw0, W_lw1, W_lw2)
    g_lws = (r1(g_lw0), r1(g_lw1), r1(g_lw2))
    b_lws = (r1(b_lw0), r1(b_lw1), r1(b_lw2))
    W_wts = (W_wt0, W_wt1, W_wt2)
    W_projs = (W_proj0, W_proj1, W_proj2)
    g_projs = (r1(g_proj0), r1(g_proj1), r1(g_proj2))
    b_projs = (r1(b_proj0), r1(b_proj1), r1(b_proj2))

    WadpP = jnp.pad(W_adp, ((0, 0), (0, C - W_adp.shape[1])))
    Wft, Wfb = W_fuse[:C], W_fuse[C:]
    G, CG = 32, 4
    g_idx = np.arange(G)[:, None, None]
    i_idx = np.arange(CG)[None, :, None]
    o_idx = np.arange(CG)[None, None, :]
    rr = np.broadcast_to(g_idx * CG + i_idx, (G, CG, CG))
    cc = np.broadcast_to(g_idx * CG + o_idx, (G, CG, CG))
    Wd1 = jnp.zeros((K, C, C), jnp.float32).at[:, rr, cc].set(W_conv1)
    Wd2 = jnp.zeros((K, C, C), jnp.float32).at[:, rr, cc].set(W_conv2)

    idx = (neighbor_idx.astype(jnp.int32)
           + (jnp.arange(K, dtype=jnp.int32) * N)[None, :]).reshape(-1)
    idx_flat = jnp.concatenate(
        [idx, jnp.zeros((N_PAD - N) * K, jnp.int32)])

    S, mu = _gram(feat)

    ps_tabs = []
    for i in range(3):
        nseg = NSEG[i]
        pw, ss, cnt = _lw(feat, clt[i], S, mu, W_lws[i], g_lws[i], b_lws[i],
                          nseg)
        y, mx = _subwt(pw, clc[i], ss, cnt, W_wts[i], nseg)
        e, es = _expseg(y, mx, clt[i], nseg)
        ps = _pf(e, es, clt[i], clc[i], feat, S, mu, W_projs[i],
                 g_projs[i], b_projs[i], nseg)
        ps_tabs.append(ps)

    y_f, st_f = _fuse1(feat, clc[0], clc[1], clc[2],
                       ps_tabs[0], ps_tabs[1], ps_tabs[2],
                       S, mu, W_proj3, r1(g_proj3), r1(b_proj3),
                       WadpP, Wft, Wfb)
    h = _fuse2(y_f, st_f, r1(g_fuse), r1(b_fuse), feat)

    z1 = _zmm_plain(h, Wd1)
    x1 = _sc_bagsum(z1, idx_flat)[:N]
    st1 = _stats(x1)
    z2 = _zmm_bn(x1, Wd2, st1, r1(g_n1), r1(b_n1))
    x2 = _sc_bagsum(z2, idx_flat)[:N]
    st2 = _stats(x2)
    return _final(x2, st2, r1(g_n2), r1(b_n2), h)
